# Initial kernel scaffold; baseline (speedup 1.0000x reference)
import jax
import jax.numpy as jnp
from jax.experimental import pallas as pl


def kernel(x, rep, in_proj_w, sconv_w, sconv_b, sconv_mod_w, sconv_mod_b, x_proj_weight, dt_projs_weight, dt_projs_bias, A_logs_w, A_logs_b, Ds_w, Ds_b, sain_gamma_w, sain_gamma_b, sain_beta_w, sain_beta_b, out_proj_w):
    raise NotImplementedError("write your pallas kernel here")


# speedup vs baseline: 12.9464x; 12.9464x over previous
import jax
import jax.numpy as jnp
from jax.experimental import pallas as pl
from jax.experimental.pallas import tpu as pltpu

L = 4096
H = 64
W = 64
DM = 96
DI = 192
NS = 16
RK = 6
K = 4
REP = 64
CH = 64
NCH = L // CH


def _pre_kernel(x_ref, repf_ref, win_ref, w9_ref, sb_ref, smw_ref, smb_ref,
                alw_ref, alb_ref, dsw_ref, dsb_ref, gw_ref, gb_ref, bw_ref,
                bb_ref, xc_ref, arow_ref, dsk_ref, gam_ref, bet_ref, pad_ref):
    f32 = jnp.float32
    repg = jnp.mean(repf_ref[...], axis=0, keepdims=True)
    dot = lambda a, b: jax.lax.dot_general(
        a, b, (((1,), (0,)), ((), ())), preferred_element_type=f32)
    s = 1.0 + dot(repg, smw_ref[...]) + smb_ref[...]
    arow_ref[...] = -jnp.exp(dot(repg, alw_ref[...]) + alb_ref[...])
    dsk_ref[...] = dot(repg, dsw_ref[...]) + dsb_ref[...]
    gam_ref[...] = dot(repg, gw_ref[...]) + gb_ref[...]
    bet_ref[...] = dot(repg, bw_ref[...]) + bb_ref[...]
    xi = dot(x_ref[...], win_ref[...]) * s
    pad_ref[...] = jnp.zeros_like(pad_ref)
    pad_ref[1:H + 1, 1:W + 1, :] = xi.reshape(H, W, DI)
    acc = jnp.zeros((H, W, DI), f32)
    for i in range(3):
        for j in range(3):
            wt = w9_ref[i * 3 + j:i * 3 + j + 1, :].reshape(1, 1, DI)
            acc = acc + wt * pad_ref[i:i + H, j:j + W, :]
    acc = acc + sb_ref[...].reshape(1, 1, DI)
    xc = acc * jax.lax.logistic(acc)
    xc_ref[...] = xc.reshape(L, DI)


def _scan_kernel(xf_ref, xb_ref, w76_ref, dtw_ref, dtb_ref, a_ref, dsk_ref,
                 yf_ref, yb_ref, delta_s, u_s, xtf_s, xtb_s, h_s):
    f32 = jnp.float32
    c = pl.program_id(1)
    xf = xf_ref[0]
    xb = xb_ref[0]
    w76 = w76_ref[0]
    dt = (((1,), (1,)), ((), ()))
    xdbl_f = jax.lax.dot_general(xf, w76, dt, preferred_element_type=f32)
    xdbl_b = jax.lax.dot_general(xb, w76, dt, preferred_element_type=f32)
    xtf_s[...] = jax.lax.dot_general(w76, xf, dt, preferred_element_type=f32)
    xtb_s[...] = jax.lax.dot_general(w76, xb, dt, preferred_element_type=f32)
    dts = jnp.concatenate([xdbl_f[:, 0:RK], xdbl_b[:, 38:38 + RK]], axis=1)
    draw = jax.lax.dot_general(dts, dtw_ref[0], (((1,), (0,)), ((), ())),
                               preferred_element_type=f32) + dtb_ref[0]
    delta_s[...] = jax.nn.softplus(draw)
    u_s[...] = delta_s[...] * jnp.concatenate([xf, xb], axis=1)

    A_f = a_ref[0, 0]
    A_b = a_ref[0, 1]
    dskf = dsk_ref[0, 0:1, :].reshape(1, DI)
    dskb = dsk_ref[0, 1:2, :].reshape(1, DI)

    @pl.when(c == 0)
    def _():
        h_s[...] = jnp.zeros_like(h_s)

    hf = h_s[0]
    hb = h_s[1]
    for s in range(CH):
        t = s
        r = CH - 1 - s
        df = delta_s[t:t + 1, 0:DI]
        uf = u_s[t:t + 1, 0:DI]
        Bf = xtf_s[RK:RK + NS, t:t + 1]
        Cf = xtf_s[RK + NS:38, t:t + 1]
        hf = jnp.exp(df * A_f) * hf + Bf * uf
        yf = jnp.sum(hf * Cf, axis=0, keepdims=True) \
            + dskf * xf_ref[0, t:t + 1, :]
        yf_ref[0, t:t + 1, :] = yf
        db = delta_s[r:r + 1, DI:2 * DI]
        ub = u_s[r:r + 1, DI:2 * DI]
        Bb = xtb_s[38 + RK:38 + RK + NS, r:r + 1]
        Cb = xtb_s[38 + RK + NS:76, r:r + 1]
        hb = jnp.exp(db * A_b) * hb + Bb * ub
        yb = jnp.sum(hb * Cb, axis=0, keepdims=True) \
            + dskb * xb_ref[0, r:r + 1, :]
        yb_ref[0, r:r + 1, :] = yb
    h_s[0] = hf
    h_s[1] = hb


def _fin_kernel(p0_ref, p1_ref, p2_ref, p3_ref, gam_ref, bet_ref, wout_ref,
                o_ref):
    y = p0_ref[...] + p1_ref[...] + p2_ref[...] + p3_ref[...]
    mu = jnp.mean(y, axis=0, keepdims=True)
    d = y - mu
    var = jnp.mean(d * d, axis=0, keepdims=True)
    yn = d * jax.lax.rsqrt(var + 1e-5)
    z = yn * gam_ref[...] + bet_ref[...]
    o_ref[...] = jnp.dot(z, wout_ref[...], preferred_element_type=jnp.float32)


def kernel(x, rep, in_proj_w, sconv_w, sconv_b, sconv_mod_w, sconv_mod_b,
           x_proj_weight, dt_projs_weight, dt_projs_bias, A_logs_w, A_logs_b,
           Ds_w, Ds_b, sain_gamma_w, sain_gamma_b, sain_beta_w, sain_beta_b,
           out_proj_w):
    f32 = jnp.float32
    sds = jax.ShapeDtypeStruct
    xflat = x.reshape(L, DM)
    repf = rep.reshape(REP, L).T

    xc, arow, dskrow, gam, bet = pl.pallas_call(
        _pre_kernel,
        out_shape=[sds((L, DI), f32), sds((1, K * DI * NS), f32),
                   sds((1, K * DI), f32), sds((1, DI), f32), sds((1, DI), f32)],
        scratch_shapes=[pltpu.VMEM((H + 2, W + 2, DI), f32)],
        compiler_params=pltpu.CompilerParams(vmem_limit_bytes=56 * 2**20),
        name="ss2d_pre",
    )(xflat, repf, in_proj_w.T, sconv_w.reshape(DI, 9).T,
      sconv_b[None, :], sconv_mod_w.T, sconv_mod_b[None, :],
      A_logs_w.T, A_logs_b[None, :], Ds_w.T, Ds_b[None, :],
      sain_gamma_w.T, sain_gamma_b[None, :], sain_beta_w.T,
      sain_beta_b[None, :])

    xcT = xc.reshape(H, W, DI).swapaxes(0, 1).reshape(L, DI)
    xin = jnp.stack([xc, xcT])
    A_T = arow.reshape(K, DI, NS).transpose(0, 2, 1)
    A_pair = jnp.stack([jnp.stack([A_T[0], A_T[2]]),
                        jnp.stack([A_T[1], A_T[3]])])
    dsk = dskrow.reshape(K, DI)
    dsk_pair = jnp.stack([jnp.stack([dsk[0], dsk[2]]),
                          jnp.stack([dsk[1], dsk[3]])])
    wx = x_proj_weight
    w76 = jnp.stack([jnp.concatenate([wx[0], wx[2]], axis=0),
                     jnp.concatenate([wx[1], wx[3]], axis=0)])
    z6 = jnp.zeros((RK, DI), f32)
    dtw = dt_projs_weight
    dtw_pair = jnp.stack([
        jnp.concatenate([jnp.concatenate([dtw[0].T, z6], axis=1),
                         jnp.concatenate([z6, dtw[2].T], axis=1)], axis=0),
        jnp.concatenate([jnp.concatenate([dtw[1].T, z6], axis=1),
                         jnp.concatenate([z6, dtw[3].T], axis=1)], axis=0),
    ])
    dtb = dt_projs_bias
    dtb_pair = jnp.stack([
        jnp.concatenate([dtb[0], dtb[2]])[None, :],
        jnp.concatenate([dtb[1], dtb[3]])[None, :],
    ])

    yf, yb = pl.pallas_call(
        _scan_kernel,
        grid=(2, NCH),
        in_specs=[
            pl.BlockSpec((1, CH, DI), lambda p, c: (p, c, 0)),
            pl.BlockSpec((1, CH, DI), lambda p, c: (p, NCH - 1 - c, 0)),
            pl.BlockSpec((1, 76, DI), lambda p, c: (p, 0, 0)),
            pl.BlockSpec((1, 2 * RK, 2 * DI), lambda p, c: (p, 0, 0)),
            pl.BlockSpec((1, 1, 2 * DI), lambda p, c: (p, 0, 0)),
            pl.BlockSpec((1, 2, NS, DI), lambda p, c: (p, 0, 0, 0)),
            pl.BlockSpec((1, 2, DI), lambda p, c: (p, 0, 0)),
        ],
        out_specs=[
            pl.BlockSpec((1, CH, DI), lambda p, c: (p, c, 0)),
            pl.BlockSpec((1, CH, DI), lambda p, c: (p, NCH - 1 - c, 0)),
        ],
        out_shape=[sds((2, L, DI), f32), sds((2, L, DI), f32)],
        scratch_shapes=[
            pltpu.VMEM((CH, 2 * DI), f32),
            pltpu.VMEM((CH, 2 * DI), f32),
            pltpu.VMEM((76, CH), f32),
            pltpu.VMEM((76, CH), f32),
            pltpu.VMEM((2, NS, DI), f32),
        ],
        compiler_params=pltpu.CompilerParams(
            dimension_semantics=(pltpu.GridDimensionSemantics.ARBITRARY,
                                 pltpu.GridDimensionSemantics.ARBITRARY),
            vmem_limit_bytes=40 * 2**20,
        ),
        name="ss2d_scan",
    )(xin, xin, w76, dtw_pair, dtb_pair, A_pair, dsk_pair)

    p2 = yf[1].reshape(W, H, DI).swapaxes(0, 1).reshape(L, DI)
    p3 = yb[1].reshape(W, H, DI).swapaxes(0, 1).reshape(L, DI)
    out = pl.pallas_call(
        _fin_kernel,
        out_shape=sds((L, DM), f32),
        compiler_params=pltpu.CompilerParams(vmem_limit_bytes=56 * 2**20),
        name="ss2d_fin",
    )(yf[0], yb[0], p2, p3, gam, bet, out_proj_w.T)
    return out.reshape(1, H, W, DM)
